# Initial kernel scaffold; baseline (speedup 1.0000x reference)
import functools

import jax
import jax.numpy as jnp
import numpy as np
from jax import lax
from jax.experimental import pallas as pl
from jax.experimental.pallas import tpu as pltpu
from jax.experimental.pallas import tpu_sc as plsc

_FIELD_DIMS = [100000] * 26
_OFFSETS = np.concatenate(([0], np.cumsum(_FIELD_DIMS)[:-1])).astype(np.int32)

_B = 4096
_F = 26
_D = 16
_NW = 32
_IDX_PER_W = _B * _F // _NW
_CHUNK = 128
_NCH = _IDX_PER_W // _CHUNK
_XO_ROWS = _B * _F // _CHUNK


def _sc_gather(emb_hbm, lin_hbm, xo_hbm, emb_out, lin_out,
               idx_v, rows_v, lin_v, sem_e, sem_l):
    nc = lax.axis_size("c")
    wid = lax.axis_index("s") * nc + lax.axis_index("c")
    base_row = wid * _NCH
    pltpu.sync_copy(xo_hbm.at[pl.ds(base_row, _NCH)], idx_v)

    def fire(j, _):
        pltpu.make_async_copy(
            emb_hbm.at[idx_v.at[j]],
            rows_v.at[pl.ds(j * _CHUNK, _CHUNK)],
            sem_e,
        ).start()
        pltpu.make_async_copy(
            lin_hbm.at[idx_v.at[j]],
            lin_v.at[j],
            sem_l,
        ).start()
        return 0

    lax.fori_loop(0, _NCH, fire, 0)

    pltpu.make_async_copy(
        emb_out.at[pl.ds(wid * _IDX_PER_W, _IDX_PER_W)], rows_v, sem_e
    ).wait()
    pltpu.make_async_copy(
        lin_out.at[pl.ds(base_row, _NCH)], lin_v, sem_l
    ).wait()

    pltpu.sync_copy(rows_v, emb_out.at[pl.ds(wid * _IDX_PER_W, _IDX_PER_W)])
    pltpu.sync_copy(lin_v, lin_out.at[pl.ds(base_row, _NCH)])


_gather_call = functools.partial(
    pl.kernel,
    out_type=[
        jax.ShapeDtypeStruct((_B * _F, _D), jnp.float32),
        jax.ShapeDtypeStruct((_XO_ROWS, _CHUNK), jnp.float32),
    ],
    mesh=plsc.VectorSubcoreMesh(core_axis_name="c", subcore_axis_name="s"),
    scratch_types=[
        pltpu.VMEM((_NCH, _CHUNK), jnp.int32),
        pltpu.VMEM((_IDX_PER_W, _D), jnp.float32),
        pltpu.VMEM((_NCH, _CHUNK), jnp.float32),
        pltpu.SemaphoreType.DMA,
        pltpu.SemaphoreType.DMA,
    ],
)


_BB = 512


def _mlp_body(e_ref, lv_ref, w1_ref, b1_ref, s1_ref, t1_ref,
              w2_ref, b2_ref, s2_ref, t2_ref,
              w3_ref, b3_ref, s3_ref, t3_ref,
              wout_ref, cout_ref, o_ref):
    h = jnp.dot(e_ref[...], w1_ref[...], preferred_element_type=jnp.float32)
    h = jnp.maximum((h + b1_ref[...]) * s1_ref[...] + t1_ref[...], 0.0)
    h = jnp.dot(h, w2_ref[...], preferred_element_type=jnp.float32)
    h = jnp.maximum((h + b2_ref[...]) * s2_ref[...] + t2_ref[...], 0.0)
    h = jnp.dot(h, w3_ref[...], preferred_element_type=jnp.float32)
    h = jnp.maximum((h + b3_ref[...]) * s3_ref[...] + t3_ref[...], 0.0)
    out = jnp.dot(h, wout_ref[...], preferred_element_type=jnp.float32)
    lr = jnp.sum(lv_ref[...], axis=1, keepdims=True)
    o_ref[...] = out + lr + cout_ref[...]


def kernel(x, lin_table, lin_bias, emb_table, W1, b1, g1, be1,
           W2, b2, g2, be2, W3, b3, g3, be3, Wout, bout):
    offsets = jnp.asarray(_OFFSETS, dtype=x.dtype)
    xo = (x + offsets[None, :]).reshape(_XO_ROWS, _CHUNK)
    lin_flat = lin_table.reshape(-1)

    e_flat, lv = _gather_call(_sc_gather)(emb_table, lin_flat, xo)
    e = e_flat.reshape(_B, _F * _D)
    lv = lv.reshape(_B, _F)

    inv = np.float32(1.0) / np.sqrt(np.float32(1.0 + 1e-5))
    s1 = (g1 * inv).reshape(1, -1)
    s2 = (g2 * inv).reshape(1, -1)
    s3 = (g3 * inv).reshape(1, -1)

    h_dim = W1.shape[1]
    full = lambda shape: pl.BlockSpec(shape, lambda i: (0, 0))
    out = pl.pallas_call(
        _mlp_body,
        grid=(_B // _BB,),
        in_specs=[
            pl.BlockSpec((_BB, _F * _D), lambda i: (i, 0)),
            pl.BlockSpec((_BB, _F), lambda i: (i, 0)),
            full((_F * _D, h_dim)), full((1, h_dim)), full((1, h_dim)), full((1, h_dim)),
            full((h_dim, h_dim)), full((1, h_dim)), full((1, h_dim)), full((1, h_dim)),
            full((h_dim, h_dim)), full((1, h_dim)), full((1, h_dim)), full((1, h_dim)),
            full((h_dim, 1)), full((1, 1)),
        ],
        out_specs=pl.BlockSpec((_BB, 1), lambda i: (i, 0)),
        out_shape=jax.ShapeDtypeStruct((_B, 1), jnp.float32),
    )(
        e, lv,
        W1, b1.reshape(1, -1), s1, be1.reshape(1, -1),
        W2, b2.reshape(1, -1), s2, be2.reshape(1, -1),
        W3, b3.reshape(1, -1), s3, be3.reshape(1, -1),
        Wout, (bout + lin_bias).reshape(1, 1),
    )
    return out


# speedup vs baseline: 1.6458x; 1.6458x over previous
import functools

import jax
import jax.numpy as jnp
import numpy as np
from jax import lax
from jax.experimental import pallas as pl
from jax.experimental.pallas import tpu as pltpu
from jax.experimental.pallas import tpu_sc as plsc

_FIELD_DIMS = [100000] * 26
_OFFSETS = np.concatenate(([0], np.cumsum(_FIELD_DIMS)[:-1])).astype(np.int32)

_B = 4096
_F = 26
_D = 16
_NW = 32
_IDX_PER_W = _B * _F // _NW
_CHUNK = 128
_NCH = _IDX_PER_W // _CHUNK
_XO_ROWS = _B * _F // _CHUNK


def _sc_gather(emb_hbm, lin_hbm, xo_hbm, emb_out, lin_out,
               idx_v, rows_v, lin_v, sem_e, sem_l):
    wid = lax.axis_index("s") * 2 + lax.axis_index("c")
    pltpu.sync_copy(xo_hbm.at[wid], idx_v)

    def fire(j, _):
        pltpu.make_async_copy(
            emb_hbm.at[idx_v.at[j]],
            rows_v.at[pl.ds(j * _CHUNK, _CHUNK)],
            sem_e,
        ).start()
        pltpu.make_async_copy(
            lin_hbm.at[idx_v.at[j]],
            lin_v.at[j],
            sem_l,
        ).start()
        return 0

    lax.fori_loop(0, _NCH, fire, 0)

    pltpu.make_async_copy(
        emb_out.at[pl.ds(wid * _IDX_PER_W, _IDX_PER_W)], rows_v, sem_e
    ).wait()
    pltpu.make_async_copy(lin_out.at[wid], lin_v, sem_l).wait()

    pltpu.sync_copy(rows_v, emb_out.at[pl.ds(wid * _IDX_PER_W, _IDX_PER_W)])
    pltpu.sync_copy(lin_v, lin_out.at[wid])


_gather_call = functools.partial(
    pl.kernel,
    out_type=[
        jax.ShapeDtypeStruct((_B * _F, _D), jnp.float32),
        jax.ShapeDtypeStruct((_NW, _NCH, _CHUNK), jnp.float32),
    ],
    mesh=plsc.VectorSubcoreMesh(
        core_axis_name="c", subcore_axis_name="s", num_cores=2, num_subcores=16
    ),
    scratch_types=[
        pltpu.VMEM((_NCH, _CHUNK), jnp.int32),
        pltpu.VMEM((_IDX_PER_W, _D), jnp.float32),
        pltpu.VMEM((_NCH, _CHUNK), jnp.float32),
        pltpu.SemaphoreType.DMA,
        pltpu.SemaphoreType.DMA,
    ],
    compiler_params=pltpu.CompilerParams(use_tc_tiling_on_sc=False),
)


_BB = 512


def _mlp_body(e_ref, lv_ref, w1_ref, b1_ref, s1_ref, t1_ref,
              w2_ref, b2_ref, s2_ref, t2_ref,
              w3_ref, b3_ref, s3_ref, t3_ref,
              wout_ref, cout_ref, o_ref):
    h = jnp.dot(e_ref[...], w1_ref[...], preferred_element_type=jnp.float32)
    h = jnp.maximum((h + b1_ref[...]) * s1_ref[...] + t1_ref[...], 0.0)
    h = jnp.dot(h, w2_ref[...], preferred_element_type=jnp.float32)
    h = jnp.maximum((h + b2_ref[...]) * s2_ref[...] + t2_ref[...], 0.0)
    h = jnp.dot(h, w3_ref[...], preferred_element_type=jnp.float32)
    h = jnp.maximum((h + b3_ref[...]) * s3_ref[...] + t3_ref[...], 0.0)
    out = jnp.dot(h, wout_ref[...], preferred_element_type=jnp.float32)
    lr = jnp.sum(lv_ref[...], axis=1, keepdims=True)
    o_ref[...] = out + lr + cout_ref[...]


def kernel(x, lin_table, lin_bias, emb_table, W1, b1, g1, be1,
           W2, b2, g2, be2, W3, b3, g3, be3, Wout, bout):
    offsets = jnp.asarray(_OFFSETS, dtype=x.dtype)
    xo = (x + offsets[None, :]).reshape(_NW, _NCH, _CHUNK)
    lin_flat = lin_table.reshape(-1)

    e_flat, lv = _gather_call(_sc_gather)(emb_table, lin_flat, xo)
    e = e_flat.reshape(_B, _F * _D)
    lv = lv.reshape(_B, _F)

    inv = np.float32(1.0) / np.sqrt(np.float32(1.0 + 1e-5))
    s1 = (g1 * inv).reshape(1, -1)
    s2 = (g2 * inv).reshape(1, -1)
    s3 = (g3 * inv).reshape(1, -1)

    h_dim = W1.shape[1]
    full = lambda shape: pl.BlockSpec(shape, lambda i: (0, 0))
    out = pl.pallas_call(
        _mlp_body,
        grid=(_B // _BB,),
        in_specs=[
            pl.BlockSpec((_BB, _F * _D), lambda i: (i, 0)),
            pl.BlockSpec((_BB, _F), lambda i: (i, 0)),
            full((_F * _D, h_dim)), full((1, h_dim)), full((1, h_dim)), full((1, h_dim)),
            full((h_dim, h_dim)), full((1, h_dim)), full((1, h_dim)), full((1, h_dim)),
            full((h_dim, h_dim)), full((1, h_dim)), full((1, h_dim)), full((1, h_dim)),
            full((h_dim, 1)), full((1, 1)),
        ],
        out_specs=pl.BlockSpec((_BB, 1), lambda i: (i, 0)),
        out_shape=jax.ShapeDtypeStruct((_B, 1), jnp.float32),
    )(
        e, lv,
        W1, b1.reshape(1, -1), s1, be1.reshape(1, -1),
        W2, b2.reshape(1, -1), s2, be2.reshape(1, -1),
        W3, b3.reshape(1, -1), s3, be3.reshape(1, -1),
        Wout, (bout + lin_bias).reshape(1, 1),
    )
    return out
